# baseline (speedup 1.0000x reference)
import jax
import jax.numpy as jnp
from jax import lax
from jax.experimental import pallas as pl
from jax.experimental.pallas import tpu as pltpu
from jax.experimental.pallas import tpu_sc as plsc

N = 16384
M = 26
C = 1000000
NC = 2
NS = 16
L = 16
NW = NC * NS
TOTAL = N * M
CHUNK = TOTAL // NW
VECS = CHUNK // L


def _tok_body(x_hbm, table_hbm, off_hbm, out_hbm, x_v, idx_v, off_v, out_v, sem):
    wid = lax.axis_index("s") * NC + lax.axis_index("c")
    base = wid * CHUNK
    pltpu.sync_copy(off_hbm, off_v)
    pltpu.sync_copy(x_hbm.at[pl.ds(base, CHUNK)], x_v)

    def body(i, carry):
        s = i * L
        xv = x_v[pl.ds(s, L)]
        off = off_v[pl.ds(lax.rem(i, 13) * L, L)]
        idx_v[pl.ds(s, L)] = xv + off
        return carry

    lax.fori_loop(0, VECS, body, 0)
    pltpu.async_copy(table_hbm.at[idx_v], out_v, sem).wait()
    pltpu.sync_copy(out_v, out_hbm.at[pl.ds(base, CHUNK)])


def kernel(x, translation, minimum):
    table = translation.reshape(-1)
    xf = x.reshape(-1)
    m208 = jnp.arange(13 * L, dtype=jnp.int32) % M
    off = m208 * C - minimum[m208]
    mesh = plsc.VectorSubcoreMesh(core_axis_name="c", subcore_axis_name="s")
    fn = pl.kernel(
        _tok_body,
        mesh=mesh,
        out_type=jax.ShapeDtypeStruct((TOTAL,), jnp.float32),
        scratch_types=[
            pltpu.VMEM((CHUNK,), jnp.int32),
            pltpu.VMEM((CHUNK,), jnp.int32),
            pltpu.VMEM((13 * L,), jnp.int32),
            pltpu.VMEM((CHUNK,), jnp.float32),
            pltpu.SemaphoreType.DMA,
        ],
    )
    out = fn(xf, table, off)
    return out.reshape(N, M)
